# baseline (speedup 1.0000x reference)
import functools

import jax
import jax.numpy as jnp
import numpy as np
from jax import lax
from jax.experimental import pallas as pl
from jax.experimental.pallas import tpu as pltpu
from jax.experimental.pallas import tpu_sc as plsc

_VOCAB = 100000
_D = 768
_B, _S = 4, 4096
_SCALE = float(np.sqrt(np.float32(_D)))

_NC = 2
_NS = 16
_NW = _NC * _NS

_POS_PER_W = _S // _NW
_CH = 32
_NCHUNK = _POS_PER_W // _CH
_NIT = _NCHUNK * _B
_CPV = _D // 16


def _make_pe(seq_len, d):
    pos = np.arange(seq_len)[:, None].astype(np.float32)
    i = np.arange(0, d, 2).astype(np.float32)
    angle = pos / np.power(10000.0, i / np.float32(d))
    pe = np.zeros((seq_len, d), dtype=np.float32)
    pe[:, 0::2] = np.sin(angle)
    pe[:, 1::2] = np.cos(angle)
    return pe


_PE_HOST = _make_pe(_S, _D)


@functools.partial(
    pl.kernel,
    out_type=jax.ShapeDtypeStruct((_B * _S, _D), jnp.float32),
    mesh=plsc.VectorSubcoreMesh(core_axis_name="c", subcore_axis_name="s"),
    scratch_types=[
        pltpu.VMEM((_B, _POS_PER_W), jnp.int32),
        pltpu.VMEM((2, _CH, _D), jnp.float32),
        pltpu.VMEM((3, _CH, _D), jnp.float32),
        pltpu.SemaphoreType.DMA,
        pltpu.SemaphoreType.DMA,
        pltpu.SemaphoreType.DMA,
        pltpu.SemaphoreType.DMA,
        pltpu.SemaphoreType.DMA,
        pltpu.SemaphoreType.DMA,
        pltpu.SemaphoreType.DMA,
        pltpu.SemaphoreType.DMA,
        pltpu.SemaphoreType.DMA,
    ],
)
def _emb_kernel(table_hbm, inp_hbm, pe_hbm, out_hbm, idx_all, pe_v, rows_v,
                isem, g0, g1, g2, s0, s1, s2, p0, p1):
    wid = lax.axis_index("s") * _NC + lax.axis_index("c")
    p_base = wid * _POS_PER_W
    gsem = [g0, g1, g2]
    ssem = [s0, s1, s2]
    psem = [p0, p1]

    def idx_copy(b):
        return pltpu.make_async_copy(
            inp_hbm.at[pl.ds(b * _S + p_base, _POS_PER_W)],
            idx_all.at[b], isem)

    def pe_copy(pc):
        return pltpu.make_async_copy(
            pe_hbm.at[pl.ds(p_base + pc * _CH, _CH)],
            pe_v.at[pc % 2], psem[pc % 2])

    def gather_copy(i):
        pc, b = divmod(i, _B)
        return pltpu.make_async_copy(
            table_hbm.at[idx_all.at[b, pl.ds(pc * _CH, _CH)]],
            rows_v.at[i % 3], gsem[i % 3])

    def store_copy(i):
        pc, b = divmod(i, _B)
        return pltpu.make_async_copy(
            rows_v.at[i % 3],
            out_hbm.at[pl.ds(b * _S + p_base + pc * _CH, _CH)],
            ssem[i % 3])

    for b in range(_B):
        idx_copy(b).start()
    pe_copy(0).start()
    for b in range(_B):
        idx_copy(b).wait()
    gather_copy(0).start()
    gather_copy(1).start()

    for i in range(_NIT):
        pc, b = divmod(i, _B)
        slot = i % 3
        gather_copy(i).wait()
        if b == 0:
            pe_copy(pc).wait()
            if pc + 1 < _NCHUNK:
                pe_copy(pc + 1).start()

        def body(r, carry):
            for c in range(_CPV):
                sl = pl.ds(c * 16, 16)
                rows_v[slot, r, sl] = (rows_v[slot, r, sl] * _SCALE
                                       + pe_v[pc % 2, r, sl])
            return carry

        lax.fori_loop(0, _CH, body, 0)

        store_copy(i).start()
        if i + 2 < _NIT:
            if i >= 1:
                store_copy(i - 1).wait()
            gather_copy(i + 2).start()

    for i in range(_NIT - 3, _NIT):
        store_copy(i).wait()


def kernel(inp, table, is_training):
    del is_training
    pe = jnp.asarray(_PE_HOST)
    out = _emb_kernel(table, inp.reshape(_B * _S), pe)
    return out.reshape(_B, _S, _D)
